# baseline (device time: 394604 ns/iter reference)
import jax
import jax.numpy as jnp
from jax import lax
from jax.experimental import pallas as pl
from jax.experimental.pallas import tpu as pltpu

N_DEV = 4
M_PER = 1024
K = 4096
N_PER = 2048
NJ = 8
NJC = N_PER // NJ


H = M_PER // 2


def _ag_gemm(x, w):
    def body(x_ref, w_ref, y_ref, amax_ref, gath_ref,
             xchunk_ref, yv_ref, copy_sems, ycopy_sems,
             send_sems, recv_sems, half_sems):
        oo = pl.program_id(0)
        j = pl.program_id(1)
        my = lax.axis_index("i")
        left = lax.rem(my + 3, N_DEV)
        right = lax.rem(my + 1, N_DEV)
        opp = lax.rem(my + 2, N_DEV)

        def full_to(tgt, sem_idx, dst_slot):
            return pltpu.make_async_remote_copy(
                src_ref=xchunk_ref.at[0],
                dst_ref=xchunk_ref.at[dst_slot],
                send_sem=send_sems.at[sem_idx],
                recv_sem=recv_sems.at[my],
                device_id=(tgt,),
                device_id_type=pl.DeviceIdType.MESH,
            )

        def fwd_right():
            return pltpu.make_async_remote_copy(
                src_ref=xchunk_ref.at[1, pl.ds(0, H), :],
                dst_ref=gath_ref.at[pl.ds(0, H), :],
                send_sem=send_sems.at[2],
                recv_sem=half_sems.at[0],
                device_id=(right,),
                device_id_type=pl.DeviceIdType.MESH,
            )

        def fwd_left():
            return pltpu.make_async_remote_copy(
                src_ref=xchunk_ref.at[2, pl.ds(H, H), :],
                dst_ref=gath_ref.at[pl.ds(H, H), :],
                send_sem=send_sems.at[3],
                recv_sem=half_sems.at[1],
                device_id=(left,),
                device_id_type=pl.DeviceIdType.MESH,
            )

        def recv_full(origin, dst_slot):
            return pltpu.make_async_remote_copy(
                src_ref=xchunk_ref.at[dst_slot],
                dst_ref=xchunk_ref.at[dst_slot],
                send_sem=send_sems.at[0],
                recv_sem=recv_sems.at[origin],
                device_id=(origin,),
                device_id_type=pl.DeviceIdType.MESH,
            )

        def recv_half(k):
            off = 0 if k == 0 else H
            return pltpu.make_async_remote_copy(
                src_ref=gath_ref.at[pl.ds(off, H), :],
                dst_ref=gath_ref.at[pl.ds(off, H), :],
                send_sem=send_sems.at[0],
                recv_sem=half_sems.at[k],
                device_id=(opp,),
                device_id_type=pl.DeviceIdType.MESH,
            )

        def chunk_copy(src, slot):
            return pltpu.make_async_copy(
                src, xchunk_ref.at[slot], copy_sems.at[slot])

        delta = jnp.where(oo == 1, 3, jnp.where(oo == 2, 1,
                          jnp.where(oo == 3, 2, 0)))
        origin = lax.rem(my + delta, N_DEV)
        slot = jnp.where(oo == 0, 0, jnp.where(oo == 2, 2, 1))

        @pl.when((oo == 0) & (j == 0))
        def _():
            cp = chunk_copy(x_ref, 0)
            cp.start()
            cp.wait()
            full_to(right, 0, 1).start()
            full_to(left, 1, 2).start()

        @pl.when((oo == 3) & (j == 0))
        def _():
            chunk_copy(gath_ref, 1).wait()

        @pl.when(j == NJ - 1)
        def _():
            @pl.when(oo == 0)
            def _():
                recv_full(left, 1).wait_recv()
                fwd_right().start()

            @pl.when(oo == 1)
            def _():
                recv_full(right, 2).wait_recv()
                fwd_left().start()

            @pl.when(oo == 2)
            def _():
                recv_half(0).wait_recv()
                recv_half(1).wait_recv()
                fwd_right().wait_send()
                chunk_copy(gath_ref, 1).start()

        yb = lax.dot_general(
            xchunk_ref[slot], w_ref[...],
            (((1,), (0,)), ((), ())),
            precision=lax.Precision.DEFAULT,
            preferred_element_type=jnp.float32,
        )
        yb = jnp.maximum(yb, 0.0)
        bmax = jnp.max(yb)

        @pl.when((oo == 0) & (j == 0))
        def _():
            amax_ref[0, 0] = bmax

        @pl.when(~((oo == 0) & (j == 0)))
        def _():
            amax_ref[0, 0] = jnp.maximum(amax_ref[0, 0], bmax)

        s = oo * NJ + j
        yslot = lax.rem(s, 2)

        def ycopy(sl):
            return pltpu.make_async_copy(
                yv_ref.at[sl],
                y_ref.at[pl.ds(origin * M_PER, M_PER),
                         pl.ds(j * NJC, NJC)],
                ycopy_sems.at[sl],
            )

        @pl.when(s >= 2)
        def _():
            ycopy(yslot).wait()

        yv_ref[yslot] = yb
        ycopy(yslot).start()

        @pl.when((oo == N_DEV - 1) & (j == NJ - 1))
        def _():
            ycopy(lax.rem(s + 1, 2)).wait()
            ycopy(yslot).wait()
            full_to(right, 0, 1).wait_send()
            full_to(left, 1, 2).wait_send()
            fwd_left().wait_send()

    return pl.pallas_call(
        body,
        grid=(N_DEV, NJ),
        in_specs=[
            pl.BlockSpec(memory_space=pl.ANY),
            pl.BlockSpec((K, NJC), lambda o, j: (0, j)),
        ],
        out_specs=[
            pl.BlockSpec(memory_space=pl.ANY),
            pl.BlockSpec(memory_space=pltpu.SMEM),
            pl.BlockSpec(memory_space=pl.ANY),
        ],
        out_shape=[
            jax.ShapeDtypeStruct((N_DEV * M_PER, N_PER), jnp.float32),
            jax.ShapeDtypeStruct((1, 1), jnp.float32),
            jax.ShapeDtypeStruct((M_PER, K), jnp.float32),
        ],
        scratch_shapes=[
            pltpu.VMEM((3, M_PER, K), jnp.float32),
            pltpu.VMEM((2, M_PER, NJC), jnp.float32),
            pltpu.SemaphoreType.DMA((2,)),
            pltpu.SemaphoreType.DMA((2,)),
            pltpu.SemaphoreType.DMA((N_DEV,)),
            pltpu.SemaphoreType.DMA((N_DEV,)),
            pltpu.SemaphoreType.DMA((2,)),
        ],
        compiler_params=pltpu.CompilerParams(
            vmem_limit_bytes=62 * 1024 * 1024),
    )(x, w)


NB = 8
MB = N_DEV * M_PER // NB


def _quant(y, local_amax):
    def body(y_ref, amax_in_ref, out_ref,
             scale_ref, abuf_ref, sbuf_ref, send_sems, recv_sems):
        b = pl.program_id(0)
        my = lax.axis_index("i")

        def peer_rdma(p):
            return pltpu.make_async_remote_copy(
                src_ref=sbuf_ref,
                dst_ref=abuf_ref.at[pl.ds(my, 1)],
                send_sem=send_sems.at[p],
                recv_sem=recv_sems.at[my],
                device_id=(p,),
                device_id_type=pl.DeviceIdType.MESH,
            )

        @pl.when(b == 0)
        def _():
            a = amax_in_ref[0, 0]
            sbuf_ref[...] = jnp.full((1, 128), a, jnp.float32)
            abuf_ref[pl.ds(my, 1), :] = jnp.full((1, 128), a, jnp.float32)
            for d in range(1, N_DEV):
                peer_rdma(lax.rem(my + d, N_DEV)).start()
            for d in range(1, N_DEV):
                p = lax.rem(my + d, N_DEV)
                recv = pltpu.make_async_remote_copy(
                    src_ref=sbuf_ref,
                    dst_ref=abuf_ref.at[pl.ds(p, 1)],
                    send_sem=send_sems.at[my],
                    recv_sem=recv_sems.at[p],
                    device_id=(p,),
                    device_id_type=pl.DeviceIdType.MESH,
                )
                recv.wait_recv()
            for d in range(1, N_DEV):
                peer_rdma(lax.rem(my + d, N_DEV)).wait_send()
            g = jnp.max(abuf_ref[...])
            scale_ref[0, 0] = jnp.maximum(g, 1e-30) / 127.0

        scale = scale_ref[0, 0]
        q = jnp.clip(jnp.round(y_ref[...] / scale), 0.0, 127.0)
        out_ref[...] = q * scale

    return pl.pallas_call(
        body,
        grid=(NB,),
        in_specs=[
            pl.BlockSpec((MB, N_PER), lambda b: (b, 0)),
            pl.BlockSpec(memory_space=pltpu.SMEM),
        ],
        out_specs=pl.BlockSpec((MB, N_PER), lambda b: (b, 0)),
        out_shape=jax.ShapeDtypeStruct((N_DEV * M_PER, N_PER), jnp.float32),
        scratch_shapes=[
            pltpu.SMEM((1, 1), jnp.float32),
            pltpu.VMEM((N_DEV, 128), jnp.float32),
            pltpu.VMEM((1, 128), jnp.float32),
            pltpu.SemaphoreType.DMA((N_DEV,)),
            pltpu.SemaphoreType.DMA((N_DEV,)),
        ],
    )(y, local_amax)


def kernel(x, w_mat):
    y, amax, _ = _ag_gemm(x, w_mat)
    return _quant(y, amax)


# device time: 278670 ns/iter; 1.4160x vs baseline; 1.4160x over previous
import jax
import jax.numpy as jnp
from jax import lax
from jax.experimental import pallas as pl
from jax.experimental.pallas import tpu as pltpu

N_DEV = 4
M_PER = 1024
K = 4096
N_PER = 2048
NJ = 4
NJC = N_PER // NJ


H = M_PER // 2


def _ag_gemm(x, w):
    def body(x_ref, w_ref, y_ref, amax_ref, gath_ref,
             xchunk_ref, yv_ref, copy_sems, ycopy_sems,
             send_sems, recv_sems, half_sems):
        oo = pl.program_id(0)
        j = pl.program_id(1)
        my = lax.axis_index("i")
        left = lax.rem(my + 3, N_DEV)
        right = lax.rem(my + 1, N_DEV)
        opp = lax.rem(my + 2, N_DEV)

        def full_to(tgt, sem_idx, dst_slot):
            return pltpu.make_async_remote_copy(
                src_ref=xchunk_ref.at[0],
                dst_ref=xchunk_ref.at[dst_slot],
                send_sem=send_sems.at[sem_idx],
                recv_sem=recv_sems.at[my],
                device_id=(tgt,),
                device_id_type=pl.DeviceIdType.MESH,
            )

        def fwd_right():
            return pltpu.make_async_remote_copy(
                src_ref=xchunk_ref.at[1, pl.ds(0, H), :],
                dst_ref=gath_ref.at[pl.ds(0, H), :],
                send_sem=send_sems.at[2],
                recv_sem=half_sems.at[0],
                device_id=(right,),
                device_id_type=pl.DeviceIdType.MESH,
            )

        def fwd_left():
            return pltpu.make_async_remote_copy(
                src_ref=xchunk_ref.at[2, pl.ds(H, H), :],
                dst_ref=gath_ref.at[pl.ds(H, H), :],
                send_sem=send_sems.at[3],
                recv_sem=half_sems.at[1],
                device_id=(left,),
                device_id_type=pl.DeviceIdType.MESH,
            )

        def recv_full(origin, dst_slot):
            return pltpu.make_async_remote_copy(
                src_ref=xchunk_ref.at[dst_slot],
                dst_ref=xchunk_ref.at[dst_slot],
                send_sem=send_sems.at[0],
                recv_sem=recv_sems.at[origin],
                device_id=(origin,),
                device_id_type=pl.DeviceIdType.MESH,
            )

        def recv_half(k):
            off = 0 if k == 0 else H
            return pltpu.make_async_remote_copy(
                src_ref=gath_ref.at[pl.ds(off, H), :],
                dst_ref=gath_ref.at[pl.ds(off, H), :],
                send_sem=send_sems.at[0],
                recv_sem=half_sems.at[k],
                device_id=(opp,),
                device_id_type=pl.DeviceIdType.MESH,
            )

        def chunk_copy(src, slot):
            return pltpu.make_async_copy(
                src, xchunk_ref.at[slot], copy_sems.at[slot])

        delta = jnp.where(oo == 1, 3, jnp.where(oo == 2, 1,
                          jnp.where(oo == 3, 2, 0)))
        origin = lax.rem(my + delta, N_DEV)
        slot = jnp.where(oo == 0, 0, jnp.where(oo == 2, 2, 1))

        @pl.when((oo == 0) & (j == 0))
        def _():
            cp = chunk_copy(x_ref, 0)
            cp.start()
            cp.wait()
            full_to(right, 0, 1).start()
            full_to(left, 1, 2).start()

        @pl.when((oo == 3) & (j == 0))
        def _():
            chunk_copy(gath_ref, 1).wait()

        @pl.when(j == NJ - 1)
        def _():
            @pl.when(oo == 0)
            def _():
                recv_full(left, 1).wait_recv()
                fwd_right().start()

            @pl.when(oo == 1)
            def _():
                recv_full(right, 2).wait_recv()
                fwd_left().start()

            @pl.when(oo == 2)
            def _():
                recv_half(0).wait_recv()
                recv_half(1).wait_recv()
                fwd_right().wait_send()
                chunk_copy(gath_ref, 1).start()

        yb = lax.dot_general(
            xchunk_ref[slot], w_ref[...],
            (((1,), (0,)), ((), ())),
            precision=lax.Precision.DEFAULT,
            preferred_element_type=jnp.float32,
        )
        yb = jnp.maximum(yb, 0.0)
        bmax = jnp.max(yb)

        @pl.when((oo == 0) & (j == 0))
        def _():
            amax_ref[0, 0] = bmax

        @pl.when(~((oo == 0) & (j == 0)))
        def _():
            amax_ref[0, 0] = jnp.maximum(amax_ref[0, 0], bmax)

        s = oo * NJ + j
        yslot = lax.rem(s, 2)

        def ycopy(sl):
            return pltpu.make_async_copy(
                yv_ref.at[sl],
                y_ref.at[pl.ds(origin * M_PER, M_PER),
                         pl.ds(j * NJC, NJC)],
                ycopy_sems.at[sl],
            )

        @pl.when(s >= 2)
        def _():
            ycopy(yslot).wait()

        yv_ref[yslot] = yb
        ycopy(yslot).start()

        @pl.when((oo == N_DEV - 1) & (j == NJ - 1))
        def _():
            ycopy(lax.rem(s + 1, 2)).wait()
            ycopy(yslot).wait()
            full_to(right, 0, 1).wait_send()
            full_to(left, 1, 2).wait_send()
            fwd_left().wait_send()

    return pl.pallas_call(
        body,
        grid=(N_DEV, NJ),
        in_specs=[
            pl.BlockSpec(memory_space=pl.ANY),
            pl.BlockSpec((K, NJC), lambda o, j: (0, j)),
        ],
        out_specs=[
            pl.BlockSpec(memory_space=pl.ANY),
            pl.BlockSpec(memory_space=pltpu.SMEM),
            pl.BlockSpec(memory_space=pl.ANY),
        ],
        out_shape=[
            jax.ShapeDtypeStruct((N_DEV * M_PER, N_PER), jnp.float32),
            jax.ShapeDtypeStruct((1, 1), jnp.float32),
            jax.ShapeDtypeStruct((M_PER, K), jnp.bfloat16),
        ],
        scratch_shapes=[
            pltpu.VMEM((3, M_PER, K), jnp.bfloat16),
            pltpu.VMEM((2, M_PER, NJC), jnp.float32),
            pltpu.SemaphoreType.DMA((2,)),
            pltpu.SemaphoreType.DMA((2,)),
            pltpu.SemaphoreType.DMA((N_DEV,)),
            pltpu.SemaphoreType.DMA((N_DEV,)),
            pltpu.SemaphoreType.DMA((2,)),
        ],
        compiler_params=pltpu.CompilerParams(
            vmem_limit_bytes=56 * 1024 * 1024),
    )(x, w)


NB = 8
MB = N_DEV * M_PER // NB


def _quant(y, local_amax):
    def body(y_ref, amax_in_ref, out_ref,
             scale_ref, abuf_ref, sbuf_ref, send_sems, recv_sems):
        b = pl.program_id(0)
        my = lax.axis_index("i")

        def peer_rdma(p):
            return pltpu.make_async_remote_copy(
                src_ref=sbuf_ref,
                dst_ref=abuf_ref.at[pl.ds(my, 1)],
                send_sem=send_sems.at[p],
                recv_sem=recv_sems.at[my],
                device_id=(p,),
                device_id_type=pl.DeviceIdType.MESH,
            )

        @pl.when(b == 0)
        def _():
            a = amax_in_ref[0, 0]
            sbuf_ref[...] = jnp.full((1, 128), a, jnp.float32)
            abuf_ref[pl.ds(my, 1), :] = jnp.full((1, 128), a, jnp.float32)
            for d in range(1, N_DEV):
                peer_rdma(lax.rem(my + d, N_DEV)).start()
            for d in range(1, N_DEV):
                p = lax.rem(my + d, N_DEV)
                recv = pltpu.make_async_remote_copy(
                    src_ref=sbuf_ref,
                    dst_ref=abuf_ref.at[pl.ds(p, 1)],
                    send_sem=send_sems.at[my],
                    recv_sem=recv_sems.at[p],
                    device_id=(p,),
                    device_id_type=pl.DeviceIdType.MESH,
                )
                recv.wait_recv()
            for d in range(1, N_DEV):
                peer_rdma(lax.rem(my + d, N_DEV)).wait_send()
            g = jnp.max(abuf_ref[...])
            scale_ref[0, 0] = jnp.maximum(g, 1e-30) / 127.0

        scale = scale_ref[0, 0]
        q = jnp.clip(jnp.round(y_ref[...] / scale), 0.0, 127.0)
        out_ref[...] = q * scale

    return pl.pallas_call(
        body,
        grid=(NB,),
        in_specs=[
            pl.BlockSpec((MB, N_PER), lambda b: (b, 0)),
            pl.BlockSpec(memory_space=pltpu.SMEM),
        ],
        out_specs=pl.BlockSpec((MB, N_PER), lambda b: (b, 0)),
        out_shape=jax.ShapeDtypeStruct((N_DEV * M_PER, N_PER), jnp.float32),
        scratch_shapes=[
            pltpu.SMEM((1, 1), jnp.float32),
            pltpu.VMEM((N_DEV, 128), jnp.float32),
            pltpu.VMEM((1, 128), jnp.float32),
            pltpu.SemaphoreType.DMA((N_DEV,)),
            pltpu.SemaphoreType.DMA((N_DEV,)),
        ],
    )(y, local_amax)


def kernel(x, w_mat):
    y, amax, _ = _ag_gemm(x.astype(jnp.bfloat16),
                          w_mat.astype(jnp.bfloat16))
    return _quant(y, amax)


# device time: 263319 ns/iter; 1.4986x vs baseline; 1.0583x over previous
import jax
import jax.numpy as jnp
from jax import lax
from jax.experimental import pallas as pl
from jax.experimental.pallas import tpu as pltpu

N_DEV = 4
M_PER = 1024
K = 4096
N_PER = 2048
NJ = 4
NJC = N_PER // NJ


H = M_PER // 2


def _ag_gemm(x, w):
    def body(x_ref, w_ref, y_ref, amax_ref, gath_ref,
             xchunk_ref, yv_ref, copy_sems, ycopy_sems,
             send_sems, recv_sems, half_sems):
        oo = pl.program_id(0)
        j = pl.program_id(1)
        my = lax.axis_index("i")
        left = lax.rem(my + 3, N_DEV)
        right = lax.rem(my + 1, N_DEV)
        opp = lax.rem(my + 2, N_DEV)

        def full_to(tgt, sem_idx, dst_slot):
            return pltpu.make_async_remote_copy(
                src_ref=xchunk_ref.at[0],
                dst_ref=xchunk_ref.at[dst_slot],
                send_sem=send_sems.at[sem_idx],
                recv_sem=recv_sems.at[my],
                device_id=(tgt,),
                device_id_type=pl.DeviceIdType.MESH,
            )

        def fwd_right():
            return pltpu.make_async_remote_copy(
                src_ref=xchunk_ref.at[1, pl.ds(0, H), :],
                dst_ref=gath_ref.at[pl.ds(0, H), :],
                send_sem=send_sems.at[2],
                recv_sem=half_sems.at[0],
                device_id=(right,),
                device_id_type=pl.DeviceIdType.MESH,
            )

        def fwd_left():
            return pltpu.make_async_remote_copy(
                src_ref=xchunk_ref.at[2, pl.ds(H, H), :],
                dst_ref=gath_ref.at[pl.ds(H, H), :],
                send_sem=send_sems.at[3],
                recv_sem=half_sems.at[1],
                device_id=(left,),
                device_id_type=pl.DeviceIdType.MESH,
            )

        def recv_full(origin, dst_slot):
            return pltpu.make_async_remote_copy(
                src_ref=xchunk_ref.at[dst_slot],
                dst_ref=xchunk_ref.at[dst_slot],
                send_sem=send_sems.at[0],
                recv_sem=recv_sems.at[origin],
                device_id=(origin,),
                device_id_type=pl.DeviceIdType.MESH,
            )

        def recv_half(k):
            off = 0 if k == 0 else H
            return pltpu.make_async_remote_copy(
                src_ref=gath_ref.at[pl.ds(off, H), :],
                dst_ref=gath_ref.at[pl.ds(off, H), :],
                send_sem=send_sems.at[0],
                recv_sem=half_sems.at[k],
                device_id=(opp,),
                device_id_type=pl.DeviceIdType.MESH,
            )

        def chunk_copy(src, slot):
            return pltpu.make_async_copy(
                src, xchunk_ref.at[slot], copy_sems.at[slot])

        delta = jnp.where(oo == 1, 3, jnp.where(oo == 2, 1,
                          jnp.where(oo == 3, 2, 0)))
        origin = lax.rem(my + delta, N_DEV)
        slot = jnp.where(oo == 0, 0, jnp.where(oo == 2, 2, 1))

        @pl.when((oo == 0) & (j == 0))
        def _():
            cp = chunk_copy(x_ref, 0)
            cp.start()
            cp.wait()
            full_to(right, 0, 1).start()
            full_to(left, 1, 2).start()

        @pl.when((oo == 3) & (j == 0))
        def _():
            chunk_copy(gath_ref, 1).wait()

        @pl.when(j == NJ - 1)
        def _():
            @pl.when(oo == 0)
            def _():
                recv_full(left, 1).wait_recv()
                fwd_right().start()

            @pl.when(oo == 1)
            def _():
                recv_full(right, 2).wait_recv()
                fwd_left().start()

            @pl.when(oo == 2)
            def _():
                recv_half(0).wait_recv()
                recv_half(1).wait_recv()
                fwd_right().wait_send()
                chunk_copy(gath_ref, 1).start()

        yb = lax.dot_general(
            xchunk_ref[slot], w_ref[...].astype(jnp.bfloat16),
            (((1,), (0,)), ((), ())),
            precision=lax.Precision.DEFAULT,
            preferred_element_type=jnp.float32,
        )
        yb = jnp.maximum(yb, 0.0)
        bmax = jnp.max(yb)

        @pl.when((oo == 0) & (j == 0))
        def _():
            amax_ref[0, 0] = bmax

        @pl.when(~((oo == 0) & (j == 0)))
        def _():
            amax_ref[0, 0] = jnp.maximum(amax_ref[0, 0], bmax)

        s = oo * NJ + j
        yslot = lax.rem(s, 2)

        def ycopy(sl):
            return pltpu.make_async_copy(
                yv_ref.at[sl],
                y_ref.at[pl.ds(origin * M_PER, M_PER),
                         pl.ds(j * NJC, NJC)],
                ycopy_sems.at[sl],
            )

        @pl.when(s >= 2)
        def _():
            ycopy(yslot).wait()

        yv_ref[yslot] = yb
        ycopy(yslot).start()

        @pl.when((oo == N_DEV - 1) & (j == NJ - 1))
        def _():
            ycopy(lax.rem(s + 1, 2)).wait()
            ycopy(yslot).wait()
            full_to(right, 0, 1).wait_send()
            full_to(left, 1, 2).wait_send()
            fwd_left().wait_send()

    return pl.pallas_call(
        body,
        grid=(N_DEV, NJ),
        in_specs=[
            pl.BlockSpec(memory_space=pl.ANY),
            pl.BlockSpec((K, NJC), lambda o, j: (0, j)),
        ],
        out_specs=[
            pl.BlockSpec(memory_space=pl.ANY),
            pl.BlockSpec(memory_space=pltpu.SMEM),
            pl.BlockSpec(memory_space=pl.ANY),
        ],
        out_shape=[
            jax.ShapeDtypeStruct((N_DEV * M_PER, N_PER), jnp.float32),
            jax.ShapeDtypeStruct((1, 1), jnp.float32),
            jax.ShapeDtypeStruct((M_PER, K), jnp.bfloat16),
        ],
        scratch_shapes=[
            pltpu.VMEM((3, M_PER, K), jnp.bfloat16),
            pltpu.VMEM((2, M_PER, NJC), jnp.float32),
            pltpu.SemaphoreType.DMA((2,)),
            pltpu.SemaphoreType.DMA((2,)),
            pltpu.SemaphoreType.DMA((N_DEV,)),
            pltpu.SemaphoreType.DMA((N_DEV,)),
            pltpu.SemaphoreType.DMA((2,)),
        ],
        compiler_params=pltpu.CompilerParams(
            vmem_limit_bytes=56 * 1024 * 1024),
    )(x, w)


NB = 8
MB = N_DEV * M_PER // NB


def _quant(y, local_amax):
    def body(y_ref, amax_in_ref, out_ref,
             scale_ref, abuf_ref, sbuf_ref, send_sems, recv_sems):
        b = pl.program_id(0)
        my = lax.axis_index("i")

        def peer_rdma(p):
            return pltpu.make_async_remote_copy(
                src_ref=sbuf_ref,
                dst_ref=abuf_ref.at[pl.ds(my, 1)],
                send_sem=send_sems.at[p],
                recv_sem=recv_sems.at[my],
                device_id=(p,),
                device_id_type=pl.DeviceIdType.MESH,
            )

        @pl.when(b == 0)
        def _():
            a = amax_in_ref[0, 0]
            sbuf_ref[...] = jnp.full((1, 128), a, jnp.float32)
            abuf_ref[pl.ds(my, 1), :] = jnp.full((1, 128), a, jnp.float32)
            for d in range(1, N_DEV):
                peer_rdma(lax.rem(my + d, N_DEV)).start()
            for d in range(1, N_DEV):
                p = lax.rem(my + d, N_DEV)
                recv = pltpu.make_async_remote_copy(
                    src_ref=sbuf_ref,
                    dst_ref=abuf_ref.at[pl.ds(p, 1)],
                    send_sem=send_sems.at[my],
                    recv_sem=recv_sems.at[p],
                    device_id=(p,),
                    device_id_type=pl.DeviceIdType.MESH,
                )
                recv.wait_recv()
            for d in range(1, N_DEV):
                peer_rdma(lax.rem(my + d, N_DEV)).wait_send()
            g = jnp.max(abuf_ref[...])
            scale_ref[0, 0] = jnp.maximum(g, 1e-30) / 127.0

        scale = scale_ref[0, 0]
        q = jnp.clip(jnp.round(y_ref[...] / scale), 0.0, 127.0)
        out_ref[...] = q * scale

    return pl.pallas_call(
        body,
        grid=(NB,),
        in_specs=[
            pl.BlockSpec((MB, N_PER), lambda b: (b, 0)),
            pl.BlockSpec(memory_space=pltpu.SMEM),
        ],
        out_specs=pl.BlockSpec((MB, N_PER), lambda b: (b, 0)),
        out_shape=jax.ShapeDtypeStruct((N_DEV * M_PER, N_PER), jnp.float32),
        scratch_shapes=[
            pltpu.SMEM((1, 1), jnp.float32),
            pltpu.VMEM((N_DEV, 128), jnp.float32),
            pltpu.VMEM((1, 128), jnp.float32),
            pltpu.SemaphoreType.DMA((N_DEV,)),
            pltpu.SemaphoreType.DMA((N_DEV,)),
        ],
        input_output_aliases={0: 0},
    )(y, local_amax)


def kernel(x, w_mat):
    y, amax, _ = _ag_gemm(x.astype(jnp.bfloat16), w_mat)
    return _quant(y, amax)
